# baseline (speedup 1.0000x reference)
import functools

import jax
import jax.numpy as jnp
from jax import lax
from jax.experimental import pallas as pl
from jax.experimental.pallas import tpu as pltpu
from jax.experimental.pallas import tpu_sc as plsc

HIDDEN = 128
EPS = 1e-12

NC = 2
NS = 16
NW = NC * NS


def _sc_gather(ids_flat, table, ch):
    n = ids_flat.shape[0]
    width = table.shape[1]
    dt = table.dtype
    per_w = n // NW
    steps = per_w // ch
    mesh = plsc.VectorSubcoreMesh(core_axis_name="c", subcore_axis_name="s")

    @functools.partial(
        pl.kernel,
        out_type=jax.ShapeDtypeStruct((n, width), dt),
        mesh=mesh,
        scratch_types=[
            pltpu.VMEM((ch,), jnp.int32),
            pltpu.VMEM((ch, width), dt),
            pltpu.SemaphoreType.DMA,
        ],
        compiler_params=pltpu.CompilerParams(use_tc_tiling_on_sc=False),
    )
    def gather_k(ids_hbm, table_hbm, out_hbm, idx_v, rows_v, sem):
        wid = lax.axis_index("s") * NC + lax.axis_index("c")

        def body(g, carry):
            base = wid * per_w + g * ch
            pltpu.sync_copy(ids_hbm.at[pl.ds(base, ch)], idx_v)
            pltpu.async_copy(table_hbm.at[idx_v], rows_v, sem).wait()
            pltpu.sync_copy(rows_v, out_hbm.at[pl.ds(base, ch)])
            return carry

        lax.fori_loop(0, steps, body, 0)

    return gather_k(ids_flat, table)


def _tc_ln_body(wg_ref, tt_ref, pos_ref, type_ref, gam_ref, bet_ref, out_ref):
    tt = tt_ref[...].astype(jnp.float32)[:, :, None]
    t0 = type_ref[0:1, :]
    dt = (type_ref[1:2, :] - t0)[None, :, :]
    wg = wg_ref[...].astype(jnp.float32)
    x = wg + pos_ref[...][None, :, :] + (t0[None, :, :] + tt * dt)
    mu = jnp.mean(x, axis=-1, keepdims=True)
    xc = x - mu
    var = jnp.mean(xc * xc, axis=-1, keepdims=True)
    y = xc * lax.rsqrt(var + EPS)
    out_ref[...] = y * gam_ref[...][None, :, :] + bet_ref[...][None, :, :]


def _tc_ln_body_acc(acc_ref, wg_ref, tt_ref, pos_ref, type_ref, gam_ref, bet_ref,
                    out_ref):
    del acc_ref
    _tc_ln_body(wg_ref, tt_ref, pos_ref, type_ref, gam_ref, bet_ref, out_ref)


def kernel(input_ids, token_type_ids, word_emb, pos_emb, type_emb, ln_gamma, ln_beta):
    b, s = input_ids.shape
    n = b * s
    ids_flat = input_ids.reshape(n).astype(jnp.int32)
    tt2 = token_type_ids.astype(jnp.int32)
    g2 = ln_gamma.reshape(1, HIDDEN)
    b2 = ln_beta.reshape(1, HIDDEN)
    v = word_emb.shape[0]
    word_bf = word_emb.astype(jnp.bfloat16)
    word_pairs = lax.bitcast_convert_type(
        word_bf.reshape(v, HIDDEN // 2, 2), jnp.int32)

    K = 4
    bc = b // K
    nk = bc * s
    nb = 32
    nblk = bc // nb

    out = None
    for k in range(K):
        wg_k = _sc_gather(ids_flat[k * nk:(k + 1) * nk], word_pairs, ch=512)
        wg_k = lax.bitcast_convert_type(wg_k, jnp.bfloat16).reshape(bc, s, HIDDEN)
        tt_k = tt2[k * bc:(k + 1) * bc]
        out_spec = pl.BlockSpec(
            (nb, s, HIDDEN), lambda i, _k=k: (_k * nblk + i, 0, 0))
        data_specs = [
            pl.BlockSpec((nb, s, HIDDEN), lambda i: (i, 0, 0)),
            pl.BlockSpec((nb, s), lambda i: (i, 0)),
            pl.BlockSpec((s, HIDDEN), lambda i: (0, 0)),
            pl.BlockSpec((2, HIDDEN), lambda i: (0, 0)),
            pl.BlockSpec((1, HIDDEN), lambda i: (0, 0)),
            pl.BlockSpec((1, HIDDEN), lambda i: (0, 0)),
        ]
        if k == 0:
            out = pl.pallas_call(
                _tc_ln_body,
                grid=(nblk,),
                in_specs=data_specs,
                out_specs=out_spec,
                out_shape=jax.ShapeDtypeStruct((b, s, HIDDEN), jnp.float32),
            )(wg_k, tt_k, pos_emb, type_emb, g2, b2)
        else:
            out = pl.pallas_call(
                _tc_ln_body_acc,
                grid=(nblk,),
                in_specs=[pl.BlockSpec(memory_space=pltpu.MemorySpace.HBM)]
                + data_specs,
                out_specs=out_spec,
                out_shape=jax.ShapeDtypeStruct((b, s, HIDDEN), jnp.float32),
                input_output_aliases={0: 0},
            )(out, wg_k, tt_k, pos_emb, type_emb, g2, b2)
    return out


# speedup vs baseline: 4.9467x; 4.9467x over previous
import functools

import jax
import jax.numpy as jnp
from jax import lax
from jax.experimental import pallas as pl
from jax.experimental.pallas import tpu as pltpu
from jax.experimental.pallas import tpu_sc as plsc

HIDDEN = 128
EPS = 1e-12

NC = 2
NS = 16
NW = NC * NS


def _sc_gather(ids_flat, table, ch):
    n = ids_flat.shape[0]
    width = table.shape[1]
    dt = table.dtype
    per_w = n // NW
    steps = per_w // ch
    mesh = plsc.VectorSubcoreMesh(core_axis_name="c", subcore_axis_name="s")

    @functools.partial(
        pl.kernel,
        out_type=jax.ShapeDtypeStruct((n, width), dt),
        mesh=mesh,
        scratch_types=[
            pltpu.VMEM((ch,), jnp.int32),
            pltpu.VMEM((ch, width), dt),
            pltpu.SemaphoreType.DMA,
        ],
    )
    def gather_k(ids_hbm, table_hbm, out_hbm, idx_v, rows_v, sem):
        wid = lax.axis_index("s") * NC + lax.axis_index("c")

        def body(g, carry):
            base = wid * per_w + g * ch
            pltpu.sync_copy(ids_hbm.at[pl.ds(base, ch)], idx_v)
            pltpu.async_copy(table_hbm.at[idx_v], rows_v, sem).wait()
            pltpu.sync_copy(rows_v, out_hbm.at[pl.ds(base, ch)])
            return carry

        lax.fori_loop(0, steps, body, 0)

    return gather_k(ids_flat, table)


def _tc_ln_body(wg_ref, tt_ref, pos_ref, type_ref, gam_ref, bet_ref, out_ref):
    tt = tt_ref[...].astype(jnp.float32)[:, :, None]
    t0 = type_ref[0:1, :]
    dt = (type_ref[1:2, :] - t0)[None, :, :]
    wg = wg_ref[...].astype(jnp.float32)
    x = wg + pos_ref[...][None, :, :] + (t0[None, :, :] + tt * dt)
    mu = jnp.mean(x, axis=-1, keepdims=True)
    xc = x - mu
    var = jnp.mean(xc * xc, axis=-1, keepdims=True)
    y = xc * lax.rsqrt(var + EPS)
    out_ref[...] = y * gam_ref[...][None, :, :] + bet_ref[...][None, :, :]


def _tc_ln_body_acc(acc_ref, wg_ref, tt_ref, pos_ref, type_ref, gam_ref, bet_ref,
                    out_ref):
    del acc_ref
    _tc_ln_body(wg_ref, tt_ref, pos_ref, type_ref, gam_ref, bet_ref, out_ref)


def kernel(input_ids, token_type_ids, word_emb, pos_emb, type_emb, ln_gamma, ln_beta):
    b, s = input_ids.shape
    n = b * s
    ids_flat = input_ids.reshape(n).astype(jnp.int32)
    tt2 = token_type_ids.astype(jnp.int32)
    g2 = ln_gamma.reshape(1, HIDDEN)
    b2 = ln_beta.reshape(1, HIDDEN)

    K = 8
    bc = b // K
    nk = bc * s
    nb = 32
    nblk = bc // nb

    out = None
    for k in range(K):
        wg_k = _sc_gather(ids_flat[k * nk:(k + 1) * nk], word_emb, ch=512)
        wg_k = wg_k.reshape(bc, s, HIDDEN)
        tt_k = tt2[k * bc:(k + 1) * bc]
        out_spec = pl.BlockSpec(
            (nb, s, HIDDEN), lambda i, _k=k: (_k * nblk + i, 0, 0))
        data_specs = [
            pl.BlockSpec((nb, s, HIDDEN), lambda i: (i, 0, 0)),
            pl.BlockSpec((nb, s), lambda i: (i, 0)),
            pl.BlockSpec((s, HIDDEN), lambda i: (0, 0)),
            pl.BlockSpec((2, HIDDEN), lambda i: (0, 0)),
            pl.BlockSpec((1, HIDDEN), lambda i: (0, 0)),
            pl.BlockSpec((1, HIDDEN), lambda i: (0, 0)),
        ]
        if k == 0:
            out = pl.pallas_call(
                _tc_ln_body,
                grid=(nblk,),
                in_specs=data_specs,
                out_specs=out_spec,
                out_shape=jax.ShapeDtypeStruct((b, s, HIDDEN), jnp.float32),
            )(wg_k, tt_k, pos_emb, type_emb, g2, b2)
        else:
            out = pl.pallas_call(
                _tc_ln_body_acc,
                grid=(nblk,),
                in_specs=[pl.BlockSpec(memory_space=pltpu.MemorySpace.HBM)]
                + data_specs,
                out_specs=out_spec,
                out_shape=jax.ShapeDtypeStruct((b, s, HIDDEN), jnp.float32),
                input_output_aliases={0: 0},
            )(out, wg_k, tt_k, pos_emb, type_emb, g2, b2)
    return out
